# baseline (speedup 1.0000x reference)
import jax
import jax.numpy as jnp
from jax import lax
from jax.experimental import pallas as pl
from jax.experimental.pallas import tpu as pltpu
from jax.experimental.pallas import tpu_sc as plsc

N = 10000
E = 320000
FEAT = 128

NC = 2
NS = 16
NW = NC * NS

EPW = E // NW
CH = 80
NCHUNK = EPW // CH
NBUF = 4

DEG_PAD = 10240
DEG_PER_TILE = DEG_PAD // NS

AGG_PER_TILE = 640
NROW_CH = AGG_PER_TILE // CH

SRC_MASK = (1 << 14) - 1

STEADY = (NCHUNK // (2 * NBUF)) * (2 * NBUF)


def _sc_body(packed_hbm, feat_hbm, agg_out, deg_out,
             shared_agg, shared_deg,
             rows_a, rows_b, rows_c, rows_d,
             pk_a, pk_b, pk_c, pk_d,
             sidx_a, sidx_b, sidx_c, sidx_d,
             didx_a, didx_b, didx_c, didx_d,
             zdeg, ones_v,
             semi_a0, semi_a1, semi_b0, semi_b1, semi_c0, semi_c1,
             semi_d0, semi_d1,
             semg_a, semg_b, semg_c, semg_d,
             semsc_a, semsc_b, semsc_c, semsc_d,
             semd_a, semd_b, semd_c, semd_d, semw):
    cid = lax.axis_index("c")
    tid = lax.axis_index("s")
    wid = cid * NS + tid

    rows = (rows_a, rows_b, rows_c, rows_d)
    pk = (pk_a, pk_b, pk_c, pk_d)
    sidx = (sidx_a, sidx_b, sidx_c, sidx_d)
    didx = (didx_a, didx_b, didx_c, didx_d)
    semi = ((semi_a0, semi_a1), (semi_b0, semi_b1),
            (semi_c0, semi_c1), (semi_d0, semi_d1))
    semg = (semg_a, semg_b, semg_c, semg_d)
    semsc = (semsc_a, semsc_b, semsc_c, semsc_d)
    semd = (semd_a, semd_b, semd_c, semd_d)

    z16 = jnp.zeros((16,), jnp.float32)
    o16 = jnp.ones((16,), jnp.float32)

    def _istart(jn, b, p):
        pltpu.async_copy(packed_hbm.at[wid, jn], pk[b].at[p], semi[b][p])

    def _iwait(b, p):
        pltpu.make_async_copy(packed_hbm.at[wid, 0], pk[b].at[p],
                              semi[b][p]).wait()

    def _unpack(b, p):
        for c in range(CH // 16):
            v = pk[b][p, pl.ds(c * 16, 16)]
            sidx[b][p, pl.ds(c * 16, 16)] = v & SRC_MASK
            didx[b][p, pl.ds(c * 16, 16)] = v >> 14

    def _gstart(b, p):
        pltpu.async_copy(feat_hbm.at[sidx[b].at[p]], rows[b], semg[b])

    def _gwait(b):
        pltpu.make_async_copy(feat_hbm.at[pl.ds(0, CH)], rows[b],
                              semg[b]).wait()

    def _scstart(b, p):
        pltpu.async_copy(rows[b], shared_agg.at[didx[b].at[p]], semsc[b],
                         add=True)

    def _scwait(b):
        pltpu.make_async_copy(feat_hbm.at[pl.ds(0, CH)], rows[b],
                              semsc[b]).wait()

    def _degstart(b, p):
        pltpu.async_copy(ones_v, shared_deg.at[didx[b].at[p]], semd[b],
                         add=True)

    def _degwait(b):
        pltpu.make_async_copy(feat_hbm.at[0, pl.ds(0, CH)], ones_v,
                              semd[b]).wait()

    for j in range(2 * NBUF):
        _istart(j, j % NBUF, j // NBUF)

    def _zero_row(i, carry):
        for c in range(FEAT // 16):
            rows_a[i, pl.ds(c * 16, 16)] = z16
        return carry

    lax.fori_loop(0, CH, _zero_row, 0)

    def _zero_deg(i, carry):
        zdeg[pl.ds(i * 16, 16)] = z16
        return carry

    lax.fori_loop(0, DEG_PER_TILE // 16, _zero_deg, 0)

    for c in range(CH // 16):
        ones_v[pl.ds(c * 16, 16)] = o16

    for k in range(NROW_CH):
        a0 = tid * AGG_PER_TILE + k * CH

        @pl.when(a0 < N)
        def _():
            pltpu.async_copy(rows_a, shared_agg.at[pl.ds(a0, CH), :], semw)
    pltpu.sync_copy(zdeg, shared_deg.at[pl.ds(tid * DEG_PER_TILE,
                                              DEG_PER_TILE)])
    for k in range(NROW_CH):
        a0 = tid * AGG_PER_TILE + k * CH

        @pl.when(a0 < N)
        def _():
            pltpu.make_async_copy(feat_hbm.at[pl.ds(0, CH)], rows_a,
                                  semw).wait()

    for b in range(NBUF):
        _iwait(b, 0)
        _unpack(b, 0)
        _gstart(b, 0)

    plsc.subcore_barrier()

    def _S(b, p):
        _gwait(b)
        _scstart(b, p)
        _degstart(b, p)

    def _T(j, b, p, refill_p, do_refill=True, do_istart=True):
        _scwait(b)
        _degwait(b)
        if do_refill:
            _iwait(b, refill_p)
            _unpack(b, refill_p)
            _gstart(b, refill_p)
        if do_istart:
            jn2 = j + 2 * NBUF

            @pl.when(jn2 < NCHUNK)
            def _():
                _istart(jn2, b, p)

    _S(0, 0)

    def _step(j8, carry):
        for q in range(2 * NBUF):
            j = j8 * (2 * NBUF) + q + 1
            _S((q + 1) % NBUF, ((q + 1) // NBUF) % 2)
            _T(j - 1, q % NBUF, (q // NBUF) % 2,
               ((q + NBUF) // NBUF) % 2)
        return carry

    lax.fori_loop(0, STEADY // (2 * NBUF), _step, 0)
    for j in range(STEADY + 1, NCHUNK):
        jp = j - 1
        _S(j % NBUF, (j // NBUF) % 2)
        _T(jp, jp % NBUF, (jp // NBUF) % 2,
           ((jp + NBUF) // NBUF) % 2,
           do_refill=(jp + NBUF) < NCHUNK, do_istart=False)
    _T(NCHUNK - 1, (NCHUNK - 1) % NBUF, ((NCHUNK - 1) // NBUF) % 2, 0,
       do_refill=False, do_istart=False)
    plsc.subcore_barrier()

    for k in range(NROW_CH):
        a0 = tid * AGG_PER_TILE + k * CH
        b = k % NBUF

        @pl.when(a0 < N)
        def _():
            if k >= NBUF:
                pltpu.make_async_copy(feat_hbm.at[pl.ds(0, CH)], rows[b],
                                      semw).wait()
            pltpu.sync_copy(shared_agg.at[pl.ds(a0, CH), :], rows[b])
            pltpu.async_copy(rows[b], agg_out.at[cid, pl.ds(a0, CH), :],
                             semw)
    for _ in range(NBUF):
        pltpu.make_async_copy(feat_hbm.at[pl.ds(0, CH)], rows_a,
                              semw).wait()
    d0 = tid * DEG_PER_TILE
    pltpu.sync_copy(shared_deg.at[pl.ds(d0, DEG_PER_TILE)], zdeg)
    pltpu.sync_copy(zdeg, deg_out.at[cid, pl.ds(d0, DEG_PER_TILE)])


@jax.jit
def _sc_aggregate(packed, feat):
    mesh = plsc.VectorSubcoreMesh(core_axis_name="c", subcore_axis_name="s",
                                  num_cores=NC, num_subcores=NS)
    idx2 = pltpu.VMEM((2, CH), jnp.int32)
    rowbuf = pltpu.VMEM((CH, FEAT), jnp.float32)
    return pl.kernel(
        _sc_body,
        out_type=[
            jax.ShapeDtypeStruct((NC, N, FEAT), jnp.float32),
            jax.ShapeDtypeStruct((NC, DEG_PAD), jnp.float32),
        ],
        mesh=mesh,
        scratch_types=(
            [pltpu.VMEM_SHARED((N, FEAT), jnp.float32),
             pltpu.VMEM_SHARED((DEG_PAD,), jnp.float32)]
            + [rowbuf] * NBUF
            + [idx2] * (3 * NBUF)
            + [pltpu.VMEM((DEG_PER_TILE,), jnp.float32),
               pltpu.VMEM((CH,), jnp.float32)]
            + [pltpu.SemaphoreType.DMA] * (2 * NBUF + 3 * NBUF + 1)
        ),
    )(packed, feat)


TC_R = 400


def _tc_body(agg_ref, deg_ref, w_ref, b_ref, out_ref):
    a = agg_ref[0] + agg_ref[1]
    d = deg_ref[0] + deg_ref[1]
    scale = 1.0 / jnp.maximum(d, 1.0)
    a = a * scale
    out_ref[...] = (
        jnp.dot(a, w_ref[...], preferred_element_type=jnp.float32)
        + b_ref[...]
    )


@jax.jit
def _tc_finish(agg_p, deg3, W, b2):
    grid = N // TC_R
    return pl.pallas_call(
        _tc_body,
        grid=(grid,),
        in_specs=[
            pl.BlockSpec((NC, TC_R, FEAT), lambda i: (0, i, 0)),
            pl.BlockSpec((NC, TC_R, 1), lambda i: (0, i, 0)),
            pl.BlockSpec((FEAT, FEAT), lambda i: (0, 0)),
            pl.BlockSpec((1, FEAT), lambda i: (0, 0)),
        ],
        out_specs=pl.BlockSpec((TC_R, FEAT), lambda i: (i, 0)),
        out_shape=jax.ShapeDtypeStruct((N, FEAT), jnp.float32),
    )(agg_p, deg3, W, b2)


def kernel(graph, feat, W, b):
    packed = (graph[0] | (graph[1] << 14)).reshape(NW, NCHUNK, CH)
    agg_p, deg_p = _sc_aggregate(packed, feat)
    return _tc_finish(agg_p, deg_p[:, :N].reshape(NC, N, 1), W,
                      b.reshape(1, FEAT))


# speedup vs baseline: 1.0313x; 1.0313x over previous
import jax
import jax.numpy as jnp
from jax import lax
from jax.experimental import pallas as pl
from jax.experimental.pallas import tpu as pltpu
from jax.experimental.pallas import tpu_sc as plsc

N = 10000
E = 320000
FEAT = 128

NC = 2
NS = 16
NW = NC * NS

EPW = E // NW
CH = 80
NCHUNK = EPW // CH
NBUF = 4

DEG_PAD = 10240
DEG_PER_TILE = DEG_PAD // NS

AGG_PER_TILE = 640
NROW_CH = AGG_PER_TILE // CH

SRC_MASK = (1 << 14) - 1

STEADY = (NCHUNK // (2 * NBUF)) * (2 * NBUF)


def _sc_body(packed_hbm, feat_hbm, agg_out, deg_out,
             shared_agg, shared_deg,
             rows_a, rows_b, rows_c, rows_d,
             pk_a, pk_b, pk_c, pk_d,
             sidx_a, sidx_b, sidx_c, sidx_d,
             didx_a, didx_b, didx_c, didx_d,
             zdeg, ones_v,
             semi_a0, semi_a1, semi_b0, semi_b1, semi_c0, semi_c1,
             semi_d0, semi_d1,
             semg_a, semg_b, semg_c, semg_d,
             semsc_a, semsc_b, semsc_c, semsc_d, semw):
    cid = lax.axis_index("c")
    tid = lax.axis_index("s")
    wid = cid * NS + tid

    rows = (rows_a, rows_b, rows_c, rows_d)
    pk = (pk_a, pk_b, pk_c, pk_d)
    sidx = (sidx_a, sidx_b, sidx_c, sidx_d)
    didx = (didx_a, didx_b, didx_c, didx_d)
    semi = ((semi_a0, semi_a1), (semi_b0, semi_b1),
            (semi_c0, semi_c1), (semi_d0, semi_d1))
    semg = (semg_a, semg_b, semg_c, semg_d)
    semsc = (semsc_a, semsc_b, semsc_c, semsc_d)

    z16 = jnp.zeros((16,), jnp.float32)
    o16 = jnp.ones((16,), jnp.float32)

    def _istart(jn, b, p):
        pltpu.async_copy(packed_hbm.at[wid, jn], pk[b].at[p], semi[b][p])

    def _iwait(b, p):
        pltpu.make_async_copy(packed_hbm.at[wid, 0], pk[b].at[p],
                              semi[b][p]).wait()

    def _unpack(b, p):
        for c in range(CH // 16):
            v = pk[b][p, pl.ds(c * 16, 16)]
            sidx[b][p, pl.ds(c * 16, 16)] = v & SRC_MASK
            didx[b][p, pl.ds(c * 16, 16)] = v >> 14

    def _gstart(b, p):
        pltpu.async_copy(feat_hbm.at[sidx[b].at[p]], rows[b], semg[b])

    def _gwait(b):
        pltpu.make_async_copy(feat_hbm.at[pl.ds(0, CH)], rows[b],
                              semg[b]).wait()

    def _scstart(b, p):
        pltpu.async_copy(rows[b], shared_agg.at[didx[b].at[p]], semsc[b],
                         add=True)

    def _scwait(b):
        pltpu.make_async_copy(feat_hbm.at[pl.ds(0, CH)], rows[b],
                              semsc[b]).wait()

    for j in range(2 * NBUF):
        _istart(j, j % NBUF, j // NBUF)

    def _zero_row(i, carry):
        for c in range(FEAT // 16):
            rows_a[i, pl.ds(c * 16, 16)] = z16
        return carry

    lax.fori_loop(0, CH, _zero_row, 0)

    def _zero_deg(i, carry):
        zdeg[pl.ds(i * 16, 16)] = z16
        return carry

    lax.fori_loop(0, DEG_PER_TILE // 16, _zero_deg, 0)

    for c in range(CH // 16):
        ones_v[pl.ds(c * 16, 16)] = o16

    for k in range(NROW_CH):
        a0 = tid * AGG_PER_TILE + k * CH

        @pl.when(a0 < N)
        def _():
            pltpu.async_copy(rows_a, shared_agg.at[pl.ds(a0, CH), :], semw)
    pltpu.sync_copy(zdeg, shared_deg.at[pl.ds(tid * DEG_PER_TILE,
                                              DEG_PER_TILE)])
    for k in range(NROW_CH):
        a0 = tid * AGG_PER_TILE + k * CH

        @pl.when(a0 < N)
        def _():
            pltpu.make_async_copy(feat_hbm.at[pl.ds(0, CH)], rows_a,
                                  semw).wait()

    for b in range(NBUF):
        _iwait(b, 0)
        _unpack(b, 0)
        _gstart(b, 0)

    plsc.subcore_barrier()

    def _consume(b, p):
        _gwait(b)
        _scstart(b, p)
        pltpu.sync_copy(ones_v, shared_deg.at[didx[b].at[p]], add=True)
        _scwait(b)

    def _step(j8, carry):
        for q in range(2 * NBUF):
            j = j8 * (2 * NBUF) + q
            b = q % NBUF
            p = q // NBUF
            _consume(b, p)
            jn = j + NBUF

            @pl.when(jn < NCHUNK)
            def _():
                _iwait(b, 1 - p)
                _unpack(b, 1 - p)
                _gstart(b, 1 - p)

            jn2 = j + 2 * NBUF

            @pl.when(jn2 < NCHUNK)
            def _():
                _istart(jn2, b, p)
        return carry

    lax.fori_loop(0, STEADY // (2 * NBUF), _step, 0)
    for j in range(STEADY, NCHUNK):
        b = j % NBUF
        p = (j // NBUF) % 2
        _consume(b, p)
        jn = j + NBUF
        if jn < NCHUNK:
            _iwait(jn % NBUF, (jn // NBUF) % 2)
            _unpack(jn % NBUF, (jn // NBUF) % 2)
            _gstart(jn % NBUF, (jn // NBUF) % 2)
    plsc.subcore_barrier()

    for k in range(NROW_CH):
        a0 = tid * AGG_PER_TILE + k * CH
        b = k % NBUF

        @pl.when(a0 < N)
        def _():
            if k >= NBUF:
                pltpu.make_async_copy(feat_hbm.at[pl.ds(0, CH)], rows[b],
                                      semw).wait()
            pltpu.sync_copy(shared_agg.at[pl.ds(a0, CH), :], rows[b])
            pltpu.async_copy(rows[b], agg_out.at[cid, pl.ds(a0, CH), :],
                             semw)
    for _ in range(NBUF):
        pltpu.make_async_copy(feat_hbm.at[pl.ds(0, CH)], rows_a,
                              semw).wait()
    d0 = tid * DEG_PER_TILE
    pltpu.sync_copy(shared_deg.at[pl.ds(d0, DEG_PER_TILE)], zdeg)
    pltpu.sync_copy(zdeg, deg_out.at[cid, pl.ds(d0, DEG_PER_TILE)])


@jax.jit
def _sc_aggregate(packed, feat):
    mesh = plsc.VectorSubcoreMesh(core_axis_name="c", subcore_axis_name="s",
                                  num_cores=NC, num_subcores=NS)
    idx2 = pltpu.VMEM((2, CH), jnp.int32)
    rowbuf = pltpu.VMEM((CH, FEAT), jnp.float32)
    return pl.kernel(
        _sc_body,
        out_type=[
            jax.ShapeDtypeStruct((NC, N, FEAT), jnp.float32),
            jax.ShapeDtypeStruct((NC, DEG_PAD), jnp.float32),
        ],
        mesh=mesh,
        scratch_types=(
            [pltpu.VMEM_SHARED((N, FEAT), jnp.float32),
             pltpu.VMEM_SHARED((DEG_PAD,), jnp.float32)]
            + [rowbuf] * NBUF
            + [idx2] * (3 * NBUF)
            + [pltpu.VMEM((DEG_PER_TILE,), jnp.float32),
               pltpu.VMEM((CH,), jnp.float32)]
            + [pltpu.SemaphoreType.DMA] * (2 * NBUF + NBUF + NBUF + 1)
        ),
    )(packed, feat)


TC_R = 400


def _tc_body(agg_ref, deg_ref, w_ref, b_ref, out_ref):
    a = agg_ref[0] + agg_ref[1]
    d = deg_ref[0] + deg_ref[1]
    scale = 1.0 / jnp.maximum(d, 1.0)
    a = a * scale
    out_ref[...] = (
        jnp.dot(a, w_ref[...], preferred_element_type=jnp.float32)
        + b_ref[...]
    )


@jax.jit
def _tc_finish(agg_p, deg3, W, b2):
    grid = N // TC_R
    return pl.pallas_call(
        _tc_body,
        grid=(grid,),
        in_specs=[
            pl.BlockSpec((NC, TC_R, FEAT), lambda i: (0, i, 0)),
            pl.BlockSpec((NC, TC_R, 1), lambda i: (0, i, 0)),
            pl.BlockSpec((FEAT, FEAT), lambda i: (0, 0)),
            pl.BlockSpec((1, FEAT), lambda i: (0, 0)),
        ],
        out_specs=pl.BlockSpec((TC_R, FEAT), lambda i: (i, 0)),
        out_shape=jax.ShapeDtypeStruct((N, FEAT), jnp.float32),
    )(agg_p, deg3, W, b2)


def kernel(graph, feat, W, b):
    packed = (graph[0] | (graph[1] << 14)).reshape(NW, NCHUNK, CH)
    agg_p, deg_p = _sc_aggregate(packed, feat)
    return _tc_finish(agg_p, deg_p[:, :N].reshape(NC, N, 1), W,
                      b.reshape(1, FEAT))


# speedup vs baseline: 1.1061x; 1.0726x over previous
import jax
import jax.numpy as jnp
from jax import lax
from jax.experimental import pallas as pl
from jax.experimental.pallas import tpu as pltpu
from jax.experimental.pallas import tpu_sc as plsc

N = 10000
E = 320000
FEAT = 128

NC = 2
NS = 16
NW = NC * NS

EPW = E // NW
CH = 80
NCHUNK = EPW // CH
NBUF = 4

DEG_PAD = 10240
DEG_PER_TILE = DEG_PAD // NS

AGG_PER_TILE = 640
NROW_CH = AGG_PER_TILE // CH

SRC_MASK = (1 << 14) - 1

STEADY = (NCHUNK // (2 * NBUF)) * (2 * NBUF)


def _sc_body(packed_hbm, feat_hbm, agg_out, deg_out,
             shared_agg, shared_deg,
             rows_a, rows_b, rows_c, rows_d,
             pk_a, pk_b, pk_c, pk_d,
             sidx_a, sidx_b, sidx_c, sidx_d,
             didx_a, didx_b, didx_c, didx_d,
             zdeg, ones_v,
             semi_a0, semi_a1, semi_b0, semi_b1, semi_c0, semi_c1,
             semi_d0, semi_d1,
             semg_a, semg_b, semg_c, semg_d,
             semsc_a, semsc_b, semsc_c, semsc_d, semw):
    cid = lax.axis_index("c")
    tid = lax.axis_index("s")
    wid = cid * NS + tid

    rows = (rows_a, rows_b, rows_c, rows_d)
    pk = (pk_a, pk_b, pk_c, pk_d)
    sidx = (sidx_a, sidx_b, sidx_c, sidx_d)
    didx = (didx_a, didx_b, didx_c, didx_d)
    semi = ((semi_a0, semi_a1), (semi_b0, semi_b1),
            (semi_c0, semi_c1), (semi_d0, semi_d1))
    semg = (semg_a, semg_b, semg_c, semg_d)
    semsc = (semsc_a, semsc_b, semsc_c, semsc_d)

    z16 = jnp.zeros((16,), jnp.float32)
    o16 = jnp.ones((16,), jnp.float32)

    def _istart(jn, b, p):
        pltpu.async_copy(packed_hbm.at[wid, jn], pk[b].at[p], semi[b][p])

    def _iwait(b, p):
        pltpu.make_async_copy(packed_hbm.at[wid, 0], pk[b].at[p],
                              semi[b][p]).wait()

    def _unpack(b, p):
        for c in range(CH // 16):
            v = pk[b][p, pl.ds(c * 16, 16)]
            sidx[b][p, pl.ds(c * 16, 16)] = v & SRC_MASK
            didx[b][p, pl.ds(c * 16, 16)] = v >> 14

    def _gstart(b, p):
        pltpu.async_copy(feat_hbm.at[sidx[b].at[p]], rows[b], semg[b])

    def _gwait(b):
        pltpu.make_async_copy(feat_hbm.at[pl.ds(0, CH)], rows[b],
                              semg[b]).wait()

    def _scstart(b, p):
        pltpu.async_copy(rows[b], shared_agg.at[didx[b].at[p]], semsc[b],
                         add=True)

    def _scwait(b):
        pltpu.make_async_copy(feat_hbm.at[pl.ds(0, CH)], rows[b],
                              semsc[b]).wait()

    for j in range(2 * NBUF):
        _istart(j, j % NBUF, j // NBUF)

    def _zero_row(i, carry):
        for c in range(FEAT // 16):
            rows_a[i, pl.ds(c * 16, 16)] = z16
        return carry

    lax.fori_loop(0, CH, _zero_row, 0)

    def _zero_deg(i, carry):
        zdeg[pl.ds(i * 16, 16)] = z16
        return carry

    lax.fori_loop(0, DEG_PER_TILE // 16, _zero_deg, 0)

    for c in range(CH // 16):
        ones_v[pl.ds(c * 16, 16)] = o16

    for k in range(NROW_CH):
        a0 = tid * AGG_PER_TILE + k * CH

        @pl.when(a0 < N)
        def _():
            pltpu.async_copy(rows_a, shared_agg.at[pl.ds(a0, CH), :], semw)
    pltpu.sync_copy(zdeg, shared_deg.at[pl.ds(tid * DEG_PER_TILE,
                                              DEG_PER_TILE)])
    for k in range(NROW_CH):
        a0 = tid * AGG_PER_TILE + k * CH

        @pl.when(a0 < N)
        def _():
            pltpu.make_async_copy(feat_hbm.at[pl.ds(0, CH)], rows_a,
                                  semw).wait()

    for b in range(NBUF):
        _iwait(b, 0)
        _unpack(b, 0)
        _gstart(b, 0)

    plsc.subcore_barrier()

    def _consume(b, p):
        _gwait(b)
        _scstart(b, p)
        pltpu.sync_copy(ones_v, shared_deg.at[didx[b].at[p]], add=True)
        _scwait(b)

    def _step(j8, carry):
        for q in range(2 * NBUF):
            j = j8 * (2 * NBUF) + q
            b = q % NBUF
            p = q // NBUF
            _consume(b, p)
            jn = j + NBUF

            @pl.when(jn < NCHUNK)
            def _():
                _iwait(b, 1 - p)
                _unpack(b, 1 - p)
                _gstart(b, 1 - p)

            jn2 = j + 2 * NBUF

            @pl.when(jn2 < NCHUNK)
            def _():
                _istart(jn2, b, p)
        return carry

    lax.fori_loop(0, STEADY // (2 * NBUF), _step, 0)
    for j in range(STEADY, NCHUNK):
        b = j % NBUF
        p = (j // NBUF) % 2
        _consume(b, p)
        jn = j + NBUF
        if jn < NCHUNK:
            _iwait(jn % NBUF, (jn // NBUF) % 2)
            _unpack(jn % NBUF, (jn // NBUF) % 2)
            _gstart(jn % NBUF, (jn // NBUF) % 2)
    plsc.subcore_barrier()

    for k in range(NROW_CH):
        a0 = tid * AGG_PER_TILE + k * CH
        b = k % NBUF

        @pl.when(a0 < N)
        def _():
            if k >= NBUF:
                pltpu.make_async_copy(feat_hbm.at[pl.ds(0, CH)], rows[b],
                                      semw).wait()
            pltpu.sync_copy(shared_agg.at[pl.ds(a0, CH), :], rows[b])
            pltpu.async_copy(rows[b], agg_out.at[cid, pl.ds(a0, CH), :],
                             semw)
    for _ in range(NBUF):
        pltpu.make_async_copy(feat_hbm.at[pl.ds(0, CH)], rows_a,
                              semw).wait()
    d0 = tid * DEG_PER_TILE
    pltpu.sync_copy(shared_deg.at[pl.ds(d0, DEG_PER_TILE)], zdeg)
    pltpu.sync_copy(zdeg, deg_out.at[cid, pl.ds(d0, DEG_PER_TILE)])


@jax.jit
def _sc_aggregate(packed, feat):
    mesh = plsc.VectorSubcoreMesh(core_axis_name="c", subcore_axis_name="s",
                                  num_cores=NC, num_subcores=NS)
    idx2 = pltpu.VMEM((2, CH), jnp.int32)
    rowbuf = pltpu.VMEM((CH, FEAT), jnp.float32)
    return pl.kernel(
        _sc_body,
        out_type=[
            jax.ShapeDtypeStruct((NC, N, FEAT), jnp.float32),
            jax.ShapeDtypeStruct((NC, DEG_PAD), jnp.float32),
        ],
        mesh=mesh,
        scratch_types=(
            [pltpu.VMEM_SHARED((N, FEAT), jnp.float32),
             pltpu.VMEM_SHARED((DEG_PAD,), jnp.float32)]
            + [rowbuf] * NBUF
            + [idx2] * (3 * NBUF)
            + [pltpu.VMEM((DEG_PER_TILE,), jnp.float32),
               pltpu.VMEM((CH,), jnp.float32)]
            + [pltpu.SemaphoreType.DMA] * (2 * NBUF + NBUF + NBUF + 1)
        ),
    )(packed, feat)


TC_R = 2000


def _tc_body(agg_ref, deg_ref, w_ref, b_ref, out_ref):
    a = agg_ref[0] + agg_ref[1]
    d = deg_ref[0] + deg_ref[1]
    scale = 1.0 / jnp.maximum(d, 1.0)
    a = a * scale
    out_ref[...] = (
        jnp.dot(a, w_ref[...], preferred_element_type=jnp.float32)
        + b_ref[...]
    )


@jax.jit
def _tc_finish(agg_p, deg3, W, b2):
    grid = N // TC_R
    return pl.pallas_call(
        _tc_body,
        grid=(grid,),
        in_specs=[
            pl.BlockSpec((NC, TC_R, FEAT), lambda i: (0, i, 0)),
            pl.BlockSpec((NC, TC_R, 1), lambda i: (0, i, 0)),
            pl.BlockSpec((FEAT, FEAT), lambda i: (0, 0)),
            pl.BlockSpec((1, FEAT), lambda i: (0, 0)),
        ],
        out_specs=pl.BlockSpec((TC_R, FEAT), lambda i: (i, 0)),
        out_shape=jax.ShapeDtypeStruct((N, FEAT), jnp.float32),
    )(agg_p, deg3, W, b2)


def kernel(graph, feat, W, b):
    packed = (graph[0] | (graph[1] << 14)).reshape(NW, NCHUNK, CH)
    agg_p, deg_p = _sc_aggregate(packed, feat)
    return _tc_finish(agg_p, deg_p[:, :N].reshape(NC, N, 1), W,
                      b.reshape(1, FEAT))


# speedup vs baseline: 1.1065x; 1.0004x over previous
import jax
import jax.numpy as jnp
from jax import lax
from jax.experimental import pallas as pl
from jax.experimental.pallas import tpu as pltpu
from jax.experimental.pallas import tpu_sc as plsc

N = 10000
E = 320000
FEAT = 128

NC = 2
NS = 16
NW = NC * NS

EPW = E // NW
CH = 80
NCHUNK = EPW // CH
NBUF = 4

DEG_PAD = 10240
DEG_PER_TILE = DEG_PAD // NS

AGG_PER_TILE = 640
NROW_CH = AGG_PER_TILE // CH

SRC_MASK = (1 << 14) - 1

STEADY = (NCHUNK // (2 * NBUF)) * (2 * NBUF)


def _sc_body(packed_hbm, feat_hbm, agg_out, deg_out,
             shared_agg, shared_deg,
             rows_a, rows_b, rows_c, rows_d,
             pk_a, pk_b, pk_c, pk_d,
             sidx_a, sidx_b, sidx_c, sidx_d,
             didx_a, didx_b, didx_c, didx_d,
             zdeg, ones_v,
             semi_a0, semi_a1, semi_b0, semi_b1, semi_c0, semi_c1,
             semi_d0, semi_d1,
             semg_a, semg_b, semg_c, semg_d,
             semsc_a, semsc_b, semsc_c, semsc_d, semw):
    cid = lax.axis_index("c")
    tid = lax.axis_index("s")
    wid = cid * NS + tid

    rows = (rows_a, rows_b, rows_c, rows_d)
    pk = (pk_a, pk_b, pk_c, pk_d)
    sidx = (sidx_a, sidx_b, sidx_c, sidx_d)
    didx = (didx_a, didx_b, didx_c, didx_d)
    semi = ((semi_a0, semi_a1), (semi_b0, semi_b1),
            (semi_c0, semi_c1), (semi_d0, semi_d1))
    semg = (semg_a, semg_b, semg_c, semg_d)
    semsc = (semsc_a, semsc_b, semsc_c, semsc_d)

    z16 = jnp.zeros((16,), jnp.float32)
    o16 = jnp.ones((16,), jnp.float32)

    def _istart(jn, b, p):
        pltpu.async_copy(packed_hbm.at[wid, jn], pk[b].at[p], semi[b][p])

    def _iwait(b, p):
        pltpu.make_async_copy(packed_hbm.at[wid, 0], pk[b].at[p],
                              semi[b][p]).wait()

    def _unpack(b, p):
        for c in range(CH // 16):
            v = pk[b][p, pl.ds(c * 16, 16)]
            sidx[b][p, pl.ds(c * 16, 16)] = v & SRC_MASK
            didx[b][p, pl.ds(c * 16, 16)] = v >> 14

    def _gstart(b, p):
        pltpu.async_copy(feat_hbm.at[sidx[b].at[p]], rows[b], semg[b])

    def _gwait(b):
        pltpu.make_async_copy(feat_hbm.at[pl.ds(0, CH)], rows[b],
                              semg[b]).wait()

    def _scstart(b, p):
        pltpu.async_copy(rows[b], shared_agg.at[didx[b].at[p]], semsc[b],
                         add=True)

    def _scwait(b):
        pltpu.make_async_copy(feat_hbm.at[pl.ds(0, CH)], rows[b],
                              semsc[b]).wait()

    for j in range(2 * NBUF):
        _istart(j, j % NBUF, j // NBUF)

    def _zero_row(i, carry):
        for c in range(FEAT // 16):
            rows_a[i, pl.ds(c * 16, 16)] = z16
        return carry

    lax.fori_loop(0, CH, _zero_row, 0)

    def _zero_deg(i, carry):
        zdeg[pl.ds(i * 16, 16)] = z16
        return carry

    lax.fori_loop(0, DEG_PER_TILE // 16, _zero_deg, 0)

    for c in range(CH // 16):
        ones_v[pl.ds(c * 16, 16)] = o16

    for k in range(NROW_CH):
        a0 = tid * AGG_PER_TILE + k * CH

        @pl.when(a0 < N)
        def _():
            pltpu.async_copy(rows_a, shared_agg.at[pl.ds(a0, CH), :], semw)
    pltpu.sync_copy(zdeg, shared_deg.at[pl.ds(tid * DEG_PER_TILE,
                                              DEG_PER_TILE)])
    for k in range(NROW_CH):
        a0 = tid * AGG_PER_TILE + k * CH

        @pl.when(a0 < N)
        def _():
            pltpu.make_async_copy(feat_hbm.at[pl.ds(0, CH)], rows_a,
                                  semw).wait()

    for b in range(NBUF):
        _iwait(b, 0)
        _unpack(b, 0)
        _gstart(b, 0)

    plsc.subcore_barrier()

    def _consume(b, p):
        _gwait(b)
        _scstart(b, p)
        pltpu.sync_copy(ones_v, shared_deg.at[didx[b].at[p]], add=True)
        _scwait(b)

    def _step(j8, carry):
        for q in range(2 * NBUF):
            j = j8 * (2 * NBUF) + q
            b = q % NBUF
            p = q // NBUF
            _consume(b, p)
            jn = j + NBUF

            @pl.when(jn < NCHUNK)
            def _():
                _iwait(b, 1 - p)
                _unpack(b, 1 - p)
                _gstart(b, 1 - p)

            jn2 = j + 2 * NBUF

            @pl.when(jn2 < NCHUNK)
            def _():
                _istart(jn2, b, p)
        return carry

    lax.fori_loop(0, STEADY // (2 * NBUF), _step, 0)
    for j in range(STEADY, NCHUNK):
        b = j % NBUF
        p = (j // NBUF) % 2
        _consume(b, p)
        jn = j + NBUF
        if jn < NCHUNK:
            _iwait(jn % NBUF, (jn // NBUF) % 2)
            _unpack(jn % NBUF, (jn // NBUF) % 2)
            _gstart(jn % NBUF, (jn // NBUF) % 2)
    plsc.subcore_barrier()

    for k in range(NROW_CH):
        a0 = tid * AGG_PER_TILE + k * CH
        b = k % NBUF

        @pl.when(a0 < N)
        def _():
            if k >= NBUF:
                pltpu.make_async_copy(feat_hbm.at[pl.ds(0, CH)], rows[b],
                                      semw).wait()
            pltpu.sync_copy(shared_agg.at[pl.ds(a0, CH), :], rows[b])
            pltpu.async_copy(rows[b], agg_out.at[cid, pl.ds(a0, CH), :],
                             semw)
    for _ in range(NBUF):
        pltpu.make_async_copy(feat_hbm.at[pl.ds(0, CH)], rows_a,
                              semw).wait()
    d0 = tid * DEG_PER_TILE
    pltpu.sync_copy(shared_deg.at[pl.ds(d0, DEG_PER_TILE)], zdeg)
    pltpu.sync_copy(zdeg, deg_out.at[cid, pl.ds(d0, DEG_PER_TILE)])


@jax.jit
def _sc_aggregate(packed, feat):
    mesh = plsc.VectorSubcoreMesh(core_axis_name="c", subcore_axis_name="s",
                                  num_cores=NC, num_subcores=NS)
    idx2 = pltpu.VMEM((2, CH), jnp.int32)
    rowbuf = pltpu.VMEM((CH, FEAT), jnp.float32)
    return pl.kernel(
        _sc_body,
        out_type=[
            jax.ShapeDtypeStruct((NC, N, FEAT), jnp.float32),
            jax.ShapeDtypeStruct((NC, DEG_PAD), jnp.float32),
        ],
        mesh=mesh,
        scratch_types=(
            [pltpu.VMEM_SHARED((N, FEAT), jnp.float32),
             pltpu.VMEM_SHARED((DEG_PAD,), jnp.float32)]
            + [rowbuf] * NBUF
            + [idx2] * (3 * NBUF)
            + [pltpu.VMEM((DEG_PER_TILE,), jnp.float32),
               pltpu.VMEM((CH,), jnp.float32)]
            + [pltpu.SemaphoreType.DMA] * (2 * NBUF + NBUF + NBUF + 1)
        ),
    )(packed, feat)


TC_R = 10000


def _tc_body(agg_ref, deg_ref, w_ref, b_ref, out_ref):
    a = agg_ref[0] + agg_ref[1]
    d = deg_ref[0] + deg_ref[1]
    scale = 1.0 / jnp.maximum(d, 1.0)
    a = a * scale
    out_ref[...] = (
        jnp.dot(a, w_ref[...], preferred_element_type=jnp.float32)
        + b_ref[...]
    )


@jax.jit
def _tc_finish(agg_p, deg3, W, b2):
    grid = N // TC_R
    return pl.pallas_call(
        _tc_body,
        grid=(grid,),
        in_specs=[
            pl.BlockSpec((NC, TC_R, FEAT), lambda i: (0, i, 0)),
            pl.BlockSpec((NC, TC_R, 1), lambda i: (0, i, 0)),
            pl.BlockSpec((FEAT, FEAT), lambda i: (0, 0)),
            pl.BlockSpec((1, FEAT), lambda i: (0, 0)),
        ],
        out_specs=pl.BlockSpec((TC_R, FEAT), lambda i: (i, 0)),
        out_shape=jax.ShapeDtypeStruct((N, FEAT), jnp.float32),
    )(agg_p, deg3, W, b2)


def kernel(graph, feat, W, b):
    packed = (graph[0] | (graph[1] << 14)).reshape(NW, NCHUNK, CH)
    agg_p, deg_p = _sc_aggregate(packed, feat)
    return _tc_finish(agg_p, deg_p[:, :N].reshape(NC, N, 1), W,
                      b.reshape(1, FEAT))
